# baseline (speedup 1.0000x reference)
import functools

import jax
import jax.numpy as jnp
from jax import lax
from jax.experimental import pallas as pl
from jax.experimental.pallas import tpu as pltpu
from jax.experimental.pallas import tpu_sc as plsc

N = 10000
E = 320000
H = 128

NC = 2
NS = 16
NW = NC * NS
EPW = E // NW
CHUNK = 80
NCHUNK = EPW // CHUNK
N_PAD = 10240
ROWS_PER_TILE = N_PAD // NS
ZROWS = 128

_f32 = jnp.float32
_bf16 = jnp.bfloat16


def _k1_body(x_ref, ws_ref, wd_ref, be1_ref, xs_ref, xd_ref):
    xb = x_ref[...]
    xs_ref[...] = jnp.dot(xb, ws_ref[...], preferred_element_type=_f32) + be1_ref[...]
    xd_ref[...] = jnp.dot(xb, wd_ref[...], preferred_element_type=_f32)


def _k3_body(e_ref, g1_ref, g2_ref, w1_ref, w2_ref, be2_ref, eout_ref, eup_ref):
    eb = e_ref[...]
    z = (jnp.dot(eb.astype(_bf16), w1_ref[...], preferred_element_type=_f32)
         + g1_ref[...] + g2_ref[...])
    h = z * jax.nn.sigmoid(z)
    up = jnp.dot(h.astype(_bf16), w2_ref[...], preferred_element_type=_f32) + be2_ref[...]
    eup_ref[...] = up
    eout_ref[...] = eb + up


def _k5_body(x_ref, p_ref, wx_ref, wa_ref, bn1_ref, wn2_ref, bn2_ref, out_ref):
    xb = x_ref[...]
    agg = p_ref[0] + p_ref[1]
    z = (jnp.dot(xb, wx_ref[...], preferred_element_type=_f32)
         + jnp.dot(agg, wa_ref[...], preferred_element_type=_f32)
         + bn1_ref[...])
    h = z * jax.nn.sigmoid(z)
    out_ref[...] = xb + jnp.dot(h, wn2_ref[...], preferred_element_type=_f32) + bn2_ref[...]


_MESH = plsc.VectorSubcoreMesh(core_axis_name="c", subcore_axis_name="s")


def _worker_id():
    return lax.axis_index("s") * NC + lax.axis_index("c")


@functools.partial(
    pl.kernel,
    out_type=[jax.ShapeDtypeStruct((E, H), _f32),
              jax.ShapeDtypeStruct((E, H), _f32)],
    mesh=_MESH,
    scratch_types=[
        pltpu.VMEM((NCHUNK, CHUNK), jnp.int32),
        pltpu.VMEM((NCHUNK, CHUNK), jnp.int32),
        pltpu.VMEM((CHUNK, H), _f32),
        pltpu.VMEM((CHUNK, H), _f32),
        pltpu.VMEM((CHUNK, H), _f32),
        pltpu.VMEM((CHUNK, H), _f32),
        pltpu.SemaphoreType.DMA,
        pltpu.SemaphoreType.DMA,
        pltpu.SemaphoreType.DMA,
        pltpu.SemaphoreType.DMA,
    ],
)
def _gather_kernel(src_hbm, dst_hbm, xs_hbm, xd_hbm, g1_hbm, g2_hbm,
                   sidx, didx, b1a, b2a, b1b, b2b, gsa, gsb, wsa, wsb):
    wid = _worker_id()
    base = wid * EPW
    pltpu.sync_copy(src_hbm.at[wid], sidx)
    pltpu.sync_copy(dst_hbm.at[wid], didx)

    def issue_gathers(j, b1, b2, sem):
        pltpu.async_copy(xs_hbm.at[sidx.at[j]], b1, sem)
        pltpu.async_copy(xd_hbm.at[didx.at[j]], b2, sem)

    def wait_gathers(j, b1, b2, sem):
        pltpu.make_async_copy(xs_hbm.at[sidx.at[j]], b1, sem).wait()
        pltpu.make_async_copy(xd_hbm.at[didx.at[j]], b2, sem).wait()

    def issue_writes(j, b1, b2, sem):
        pltpu.async_copy(b1, g1_hbm.at[pl.ds(base + j * CHUNK, CHUNK)], sem)
        pltpu.async_copy(b2, g2_hbm.at[pl.ds(base + j * CHUNK, CHUNK)], sem)

    def wait_writes(j, b1, b2, sem):
        pltpu.make_async_copy(b1, g1_hbm.at[pl.ds(base + j * CHUNK, CHUNK)], sem).wait()
        pltpu.make_async_copy(b2, g2_hbm.at[pl.ds(base + j * CHUNK, CHUNK)], sem).wait()

    issue_gathers(0, b1a, b2a, gsa)

    def step(t, carry):
        c0 = 2 * t
        c1 = c0 + 1
        c2 = c0 + 2
        wait_gathers(c0, b1a, b2a, gsa)
        issue_gathers(c1, b1b, b2b, gsb)
        issue_writes(c0, b1a, b2a, wsa)
        wait_gathers(c1, b1b, b2b, gsb)
        wait_writes(c0, b1a, b2a, wsa)
        issue_gathers(c2, b1a, b2a, gsa)
        issue_writes(c1, b1b, b2b, wsb)
        wait_writes(c1, b1b, b2b, wsb)
        return carry

    lax.fori_loop(0, (NCHUNK - 1) // 2, step, 0)
    last = NCHUNK - 1
    wait_gathers(last, b1a, b2a, gsa)
    issue_writes(last, b1a, b2a, wsa)
    wait_writes(last, b1a, b2a, wsa)


@functools.partial(
    pl.kernel,
    out_type=jax.ShapeDtypeStruct((NC, N_PAD, H), _f32),
    mesh=_MESH,
    scratch_types=[
        pltpu.VMEM((NCHUNK, CHUNK), jnp.int32),
        pltpu.VMEM((CHUNK, H), _f32),
        pltpu.VMEM((ZROWS, H), _f32),
        pltpu.VMEM_SHARED((N_PAD, H), _f32),
    ],
)
def _scatter_kernel(dst_hbm, eup_hbm, part_hbm, didx, ebuf, zbuf, agg):
    cid = lax.axis_index("c")
    sid = lax.axis_index("s")
    wid = _worker_id()
    base = wid * EPW

    pltpu.sync_copy(dst_hbm.at[wid], didx)

    zeros16 = jnp.zeros((16,), _f32)

    def zrow(i, carry):
        for c in range(H // 16):
            zbuf[i, pl.ds(c * 16, 16)] = zeros16
        return carry

    lax.fori_loop(0, ZROWS, zrow, 0)
    for rep in range(ROWS_PER_TILE // ZROWS):
        pltpu.sync_copy(zbuf, agg.at[pl.ds(sid * ROWS_PER_TILE + rep * ZROWS, ZROWS)])
    plsc.subcore_barrier()

    def step(j, carry):
        pltpu.sync_copy(eup_hbm.at[pl.ds(base + j * CHUNK, CHUNK)], ebuf)
        pltpu.sync_copy(ebuf, agg.at[didx.at[j]], add=True)
        return carry

    lax.fori_loop(0, NCHUNK, step, 0)
    plsc.subcore_barrier()
    pltpu.sync_copy(agg.at[pl.ds(sid * ROWS_PER_TILE, ROWS_PER_TILE)],
                    part_hbm.at[cid, pl.ds(sid * ROWS_PER_TILE, ROWS_PER_TILE)])


def kernel(x, edge_index, e, We1, be1, We2, be2, Wn1, bn1, Wn2, bn2):
    src = edge_index[0].astype(jnp.int32).reshape(NW, NCHUNK, CHUNK)
    dst = edge_index[1].astype(jnp.int32).reshape(NW, NCHUNK, CHUNK)

    W1e, W1s, W1d = We1[:H], We1[H:2 * H], We1[2 * H:]
    Wn1x, Wn1a = Wn1[:H], Wn1[H:]
    be1_2d = be1.reshape(1, H)
    be2_2d = be2.reshape(1, H)
    bn1_2d = bn1.reshape(1, H)
    bn2_2d = bn2.reshape(1, H)

    xs_proj, xd_proj = pl.pallas_call(
        _k1_body,
        out_shape=[jax.ShapeDtypeStruct((N, H), _f32),
                   jax.ShapeDtypeStruct((N, H), _f32)],
    )(x, W1s, W1d, be1_2d)

    g1, g2 = _gather_kernel(src, dst, xs_proj, xd_proj)

    BE = 2000
    grid = E // BE
    e_out, e_up = pl.pallas_call(
        _k3_body,
        grid=(grid,),
        in_specs=[
            pl.BlockSpec((BE, H), lambda i: (i, 0)),
            pl.BlockSpec((BE, H), lambda i: (i, 0)),
            pl.BlockSpec((BE, H), lambda i: (i, 0)),
            pl.BlockSpec((H, H), lambda i: (0, 0)),
            pl.BlockSpec((H, H), lambda i: (0, 0)),
            pl.BlockSpec((1, H), lambda i: (0, 0)),
        ],
        out_specs=[
            pl.BlockSpec((BE, H), lambda i: (i, 0)),
            pl.BlockSpec((BE, H), lambda i: (i, 0)),
        ],
        out_shape=[jax.ShapeDtypeStruct((E, H), _f32),
                   jax.ShapeDtypeStruct((E, H), _f32)],
    )(e, g1, g2, W1e.astype(_bf16), We2.astype(_bf16), be2_2d)

    partials = _scatter_kernel(dst, e_up)[:, :N, :]

    x_out = pl.pallas_call(
        _k5_body,
        out_shape=jax.ShapeDtypeStruct((N, H), _f32),
    )(x, partials, Wn1x, Wn1a, bn1_2d, Wn2, bn2_2d)

    return (x_out, e_out)


# speedup vs baseline: 1.1157x; 1.1157x over previous
import functools

import jax
import jax.numpy as jnp
from jax import lax
from jax.experimental import pallas as pl
from jax.experimental.pallas import tpu as pltpu
from jax.experimental.pallas import tpu_sc as plsc

N = 10000
E = 320000
H = 128

NC = 2
NS = 16
NW = NC * NS
EPW = E // NW
CHUNK = 80
NCHUNK = EPW // CHUNK
N_PAD = 10240
ROWS_PER_TILE = N_PAD // NS

_f32 = jnp.float32
_bf16 = jnp.bfloat16


def _k1_body(x_ref, ws_ref, wd_ref, be1_ref, xs_ref, xd_ref):
    xb = x_ref[...]
    xs_ref[...] = jnp.dot(xb, ws_ref[...], preferred_element_type=_f32) + be1_ref[...]
    xd_ref[...] = jnp.dot(xb, wd_ref[...], preferred_element_type=_f32)


def _k3_body(e_ref, g1_ref, g2_ref, w1_ref, w2_ref, be2_ref, eout_ref, eup_ref):
    eb = e_ref[...]
    z = (jnp.dot(eb.astype(_bf16), w1_ref[...], preferred_element_type=_f32)
         + g1_ref[...] + g2_ref[...])
    h = z * jax.nn.sigmoid(z)
    up = jnp.dot(h.astype(_bf16), w2_ref[...], preferred_element_type=_f32) + be2_ref[...]
    eup_ref[...] = up
    eout_ref[...] = eb + up


def _k5_body(x_ref, p_ref, wx_ref, wa_ref, bn1_ref, wn2_ref, bn2_ref, out_ref):
    xb = x_ref[...]
    agg = p_ref[0] + p_ref[1]
    z = (jnp.dot(xb, wx_ref[...], preferred_element_type=_f32)
         + jnp.dot(agg, wa_ref[...], preferred_element_type=_f32)
         + bn1_ref[...])
    h = z * jax.nn.sigmoid(z)
    out_ref[...] = xb + jnp.dot(h, wn2_ref[...], preferred_element_type=_f32) + bn2_ref[...]


_MESH = plsc.VectorSubcoreMesh(core_axis_name="c", subcore_axis_name="s")


def _worker_id():
    return lax.axis_index("s") * NC + lax.axis_index("c")


@functools.partial(
    pl.kernel,
    out_type=[jax.ShapeDtypeStruct((E, H), _f32),
              jax.ShapeDtypeStruct((E, H), _f32)],
    mesh=_MESH,
    scratch_types=[
        pltpu.VMEM((NCHUNK, CHUNK), jnp.int32),
        pltpu.VMEM((NCHUNK, CHUNK), jnp.int32),
        pltpu.VMEM((CHUNK, H), _f32),
        pltpu.VMEM((CHUNK, H), _f32),
        pltpu.VMEM((CHUNK, H), _f32),
        pltpu.VMEM((CHUNK, H), _f32),
        pltpu.SemaphoreType.DMA,
        pltpu.SemaphoreType.DMA,
        pltpu.SemaphoreType.DMA,
        pltpu.SemaphoreType.DMA,
    ],
)
def _gather_kernel(src_hbm, dst_hbm, xs_hbm, xd_hbm, g1_hbm, g2_hbm,
                   sidx, didx, b1a, b2a, b1b, b2b, gsa, gsb, wsa, wsb):
    wid = _worker_id()
    base = wid * EPW
    pltpu.sync_copy(src_hbm.at[wid], sidx)
    pltpu.sync_copy(dst_hbm.at[wid], didx)

    def issue_gathers(j, b1, b2, sem):
        pltpu.async_copy(xs_hbm.at[sidx.at[j]], b1, sem)
        pltpu.async_copy(xd_hbm.at[didx.at[j]], b2, sem)

    def wait_gathers(j, b1, b2, sem):
        pltpu.make_async_copy(xs_hbm.at[sidx.at[j]], b1, sem).wait()
        pltpu.make_async_copy(xd_hbm.at[didx.at[j]], b2, sem).wait()

    def issue_writes(j, b1, b2, sem):
        pltpu.async_copy(b1, g1_hbm.at[pl.ds(base + j * CHUNK, CHUNK)], sem)
        pltpu.async_copy(b2, g2_hbm.at[pl.ds(base + j * CHUNK, CHUNK)], sem)

    def wait_writes(j, b1, b2, sem):
        pltpu.make_async_copy(b1, g1_hbm.at[pl.ds(base + j * CHUNK, CHUNK)], sem).wait()
        pltpu.make_async_copy(b2, g2_hbm.at[pl.ds(base + j * CHUNK, CHUNK)], sem).wait()

    issue_gathers(0, b1a, b2a, gsa)

    def step(t, carry):
        c0 = 2 * t
        c1 = c0 + 1
        c2 = c0 + 2
        wait_gathers(c0, b1a, b2a, gsa)
        issue_gathers(c1, b1b, b2b, gsb)
        issue_writes(c0, b1a, b2a, wsa)
        wait_gathers(c1, b1b, b2b, gsb)
        wait_writes(c0, b1a, b2a, wsa)
        issue_gathers(c2, b1a, b2a, gsa)
        issue_writes(c1, b1b, b2b, wsb)
        wait_writes(c1, b1b, b2b, wsb)
        return carry

    lax.fori_loop(0, (NCHUNK - 1) // 2, step, 0)
    last = NCHUNK - 1
    wait_gathers(last, b1a, b2a, gsa)
    issue_writes(last, b1a, b2a, wsa)
    wait_writes(last, b1a, b2a, wsa)


@functools.partial(
    pl.kernel,
    out_type=jax.ShapeDtypeStruct((NC, N_PAD, H), _f32),
    mesh=_MESH,
    scratch_types=[
        pltpu.VMEM((NCHUNK, CHUNK), jnp.int32),
        pltpu.VMEM((CHUNK, H), _f32),
        pltpu.VMEM((CHUNK, H), _f32),
        pltpu.VMEM_SHARED((N_PAD, H), _f32),
        pltpu.SemaphoreType.DMA,
        pltpu.SemaphoreType.DMA,
    ],
)
def _scatter_kernel(dst_hbm, eup_hbm, part_hbm, didx, ebufa, ebufb, agg,
                    lsa, lsb):
    cid = lax.axis_index("c")
    sid = lax.axis_index("s")
    wid = _worker_id()
    base = wid * EPW

    pltpu.sync_copy(dst_hbm.at[wid], didx)

    zeros16 = jnp.zeros((16,), _f32)

    def zrow(i, carry):
        for c in range(H // 16):
            ebufa[i, pl.ds(c * 16, 16)] = zeros16
        return carry

    lax.fori_loop(0, CHUNK, zrow, 0)
    for rep in range(ROWS_PER_TILE // CHUNK):
        pltpu.sync_copy(ebufa, agg.at[pl.ds(sid * ROWS_PER_TILE + rep * CHUNK, CHUNK)])
    plsc.subcore_barrier()

    def issue_load(j, buf, sem):
        pltpu.async_copy(eup_hbm.at[pl.ds(base + j * CHUNK, CHUNK)], buf, sem)

    def wait_load(j, buf, sem):
        pltpu.make_async_copy(eup_hbm.at[pl.ds(base + j * CHUNK, CHUNK)], buf, sem).wait()

    issue_load(0, ebufa, lsa)

    def step(t, carry):
        c0 = 2 * t
        c1 = c0 + 1
        c2 = c0 + 2
        wait_load(c0, ebufa, lsa)
        issue_load(c1, ebufb, lsb)
        pltpu.sync_copy(ebufa, agg.at[didx.at[c0]], add=True)
        issue_load(c2, ebufa, lsa)
        wait_load(c1, ebufb, lsb)
        pltpu.sync_copy(ebufb, agg.at[didx.at[c1]], add=True)
        return carry

    lax.fori_loop(0, (NCHUNK - 1) // 2, step, 0)
    last = NCHUNK - 1
    wait_load(last, ebufa, lsa)
    pltpu.sync_copy(ebufa, agg.at[didx.at[last]], add=True)
    plsc.subcore_barrier()
    pltpu.sync_copy(agg.at[pl.ds(sid * ROWS_PER_TILE, ROWS_PER_TILE)],
                    part_hbm.at[cid, pl.ds(sid * ROWS_PER_TILE, ROWS_PER_TILE)])


def kernel(x, edge_index, e, We1, be1, We2, be2, Wn1, bn1, Wn2, bn2):
    src = edge_index[0].astype(jnp.int32).reshape(NW, NCHUNK, CHUNK)
    dst = edge_index[1].astype(jnp.int32).reshape(NW, NCHUNK, CHUNK)

    W1e, W1s, W1d = We1[:H], We1[H:2 * H], We1[2 * H:]
    Wn1x, Wn1a = Wn1[:H], Wn1[H:]
    be1_2d = be1.reshape(1, H)
    be2_2d = be2.reshape(1, H)
    bn1_2d = bn1.reshape(1, H)
    bn2_2d = bn2.reshape(1, H)

    xs_proj, xd_proj = pl.pallas_call(
        _k1_body,
        out_shape=[jax.ShapeDtypeStruct((N, H), _f32),
                   jax.ShapeDtypeStruct((N, H), _f32)],
    )(x, W1s, W1d, be1_2d)

    g1, g2 = _gather_kernel(src, dst, xs_proj, xd_proj)

    BE = 2000
    grid = E // BE
    e_out, e_up = pl.pallas_call(
        _k3_body,
        grid=(grid,),
        in_specs=[
            pl.BlockSpec((BE, H), lambda i: (i, 0)),
            pl.BlockSpec((BE, H), lambda i: (i, 0)),
            pl.BlockSpec((BE, H), lambda i: (i, 0)),
            pl.BlockSpec((H, H), lambda i: (0, 0)),
            pl.BlockSpec((H, H), lambda i: (0, 0)),
            pl.BlockSpec((1, H), lambda i: (0, 0)),
        ],
        out_specs=[
            pl.BlockSpec((BE, H), lambda i: (i, 0)),
            pl.BlockSpec((BE, H), lambda i: (i, 0)),
        ],
        out_shape=[jax.ShapeDtypeStruct((E, H), _f32),
                   jax.ShapeDtypeStruct((E, H), _f32)],
    )(e, g1, g2, W1e.astype(_bf16), We2.astype(_bf16), be2_2d)

    partials = _scatter_kernel(dst, e_up)[:, :N, :]

    x_out = pl.pallas_call(
        _k5_body,
        out_shape=jax.ShapeDtypeStruct((N, H), _f32),
    )(x, partials, Wn1x, Wn1a, bn1_2d, Wn2, bn2_2d)

    return (x_out, e_out)
